# Initial kernel scaffold; baseline (speedup 1.0000x reference)
import jax
import jax.numpy as jnp
from jax.experimental import pallas as pl


def kernel(x, edge_index, edge_weight, W1, b1, W2, b2):
    raise NotImplementedError("write your pallas kernel here")


# speedup vs baseline: 2.4275x; 2.4275x over previous
import functools

import jax
import jax.numpy as jnp
from jax.experimental import pallas as pl
from jax.experimental.pallas import tpu as pltpu

N = 10000
E = 160000
D = 256
H = 16
C = 64

NPAD = 10240
BLK = 1024


def _mm1_body(x_ref, w_ref, o_ref):
    o_ref[...] = jnp.dot(x_ref[...], w_ref[...], preferred_element_type=jnp.float32)


def _matmul(x, w):
    n, d = x.shape
    f = w.shape[1]
    fo = max(f, 128)
    wpad = jnp.zeros((d, fo), w.dtype).at[:, :f].set(w)
    grid = (n // BLK,)
    out = pl.pallas_call(
        _mm1_body,
        grid=grid,
        in_specs=[
            pl.BlockSpec((BLK, d), lambda i: (i, 0)),
            pl.BlockSpec((d, fo), lambda i: (0, 0)),
        ],
        out_specs=pl.BlockSpec((BLK, fo), lambda i: (i, 0)),
        out_shape=jax.ShapeDtypeStruct((n, fo), jnp.float32),
    )(x, wpad)
    return out[:, :f]


def _logsoftmax_body(x_ref, o_ref):
    x = x_ref[...]
    m = jnp.max(x, axis=1, keepdims=True)
    ex = jnp.exp(x - m)
    s = jnp.sum(ex, axis=1, keepdims=True)
    o_ref[...] = x - m - jnp.log(s)


def _log_softmax(x):
    n, f = x.shape
    grid = (n // BLK,)
    return pl.pallas_call(
        _logsoftmax_body,
        grid=grid,
        in_specs=[pl.BlockSpec((BLK, f), lambda i: (i, 0))],
        out_specs=pl.BlockSpec((BLK, f), lambda i: (i, 0)),
        out_shape=jax.ShapeDtypeStruct((n, f), jnp.float32),
    )(x)


def kernel(x, edge_index, edge_weight, W1, b1, W2, b2):
    n = x.shape[0]
    src = edge_index[0]
    dst = edge_index[1]
    ew = edge_weight

    deg = jnp.ones((n,), jnp.float32).at[dst].add(ew)
    dinv = jax.lax.rsqrt(deg)

    npad = ((n + BLK - 1) // BLK) * BLK
    xpad = jnp.zeros((npad, D), x.dtype).at[:n].set(x)

    xw1 = _matmul(xpad, W1)[:n]
    xs1 = xw1 * dinv[:, None]
    agg1 = jnp.zeros((n, H), jnp.float32).at[dst].add(xs1[src] * ew[:, None])
    h = jax.nn.relu(agg1 * dinv[:, None] + xw1 * (dinv * dinv)[:, None] + b1)

    hpad = jnp.zeros((npad, H), h.dtype).at[:n].set(h)
    xw2 = _matmul(hpad, W2)[:n]
    xs2 = xw2 * dinv[:, None]
    agg2 = jnp.zeros((n, C), jnp.float32).at[dst].add(xs2[src] * ew[:, None])
    o = agg2 * dinv[:, None] + xw2 * (dinv * dinv)[:, None] + b2

    opad = jnp.zeros((npad, C), o.dtype).at[:n].set(o)
    return _log_softmax(opad)[:n]


# speedup vs baseline: 15.9724x; 6.5798x over previous
import functools

import jax
import jax.numpy as jnp
from jax import lax
from jax.experimental import pallas as pl
from jax.experimental.pallas import tpu as pltpu
from jax.experimental.pallas import tpu_sc as plsc

N = 10000
E = 160000
D = 256
H = 16
C = 64

BLK = 1024
NACC = 10240
KB = 128
NT = 32
EPT = 5120
NCHUNK = EPT // KB
RPT = NACC // 16
EPAD = NT * EPT


def _sc_mesh():
    return plsc.VectorSubcoreMesh(core_axis_name="c", subcore_axis_name="s")


def _deg_body(ewb, dstb, zeros_hbm, out_hbm, dst_v, ew_v, acc):
    c = lax.axis_index("c")
    s = lax.axis_index("s")
    wid = s * 2 + c
    pltpu.sync_copy(dstb.at[wid], dst_v)
    pltpu.sync_copy(ewb.at[wid], ew_v)
    base = s * RPT
    pltpu.sync_copy(zeros_hbm.at[pl.ds(base, RPT)], acc.at[pl.ds(base, RPT)])
    plsc.subcore_barrier()

    def body(j, carry):
        pltpu.sync_copy(ew_v.at[j], acc.at[dst_v.at[j]], add=True)
        return carry

    lax.fori_loop(0, NCHUNK, body, 0)
    plsc.subcore_barrier()
    pltpu.sync_copy(acc.at[pl.ds(base, RPT)], out_hbm.at[c, pl.ds(base, RPT)])


_deg_kernel = functools.partial(
    pl.kernel,
    out_type=jax.ShapeDtypeStruct((2, NACC), jnp.float32),
    mesh=_sc_mesh(),
    scratch_types=[
        pltpu.VMEM((NCHUNK, KB), jnp.int32),
        pltpu.VMEM((NCHUNK, KB), jnp.float32),
        pltpu.VMEM_SHARED((NACC,), jnp.float32),
    ],
    compiler_params=pltpu.CompilerParams(use_tc_tiling_on_sc=False),
)(_deg_body)


def _make_agg(F):
    def body(table_hbm, srcb, dstb, ewb, out_hbm, src_v, dst_v, ew_v, rows_v,
             acc, sem0, sem1):
        c = lax.axis_index("c")
        s = lax.axis_index("s")
        wid = s * 2 + c
        pltpu.sync_copy(srcb.at[wid], src_v)
        pltpu.sync_copy(dstb.at[wid], dst_v)
        pltpu.sync_copy(ewb.at[wid], ew_v)
        base = s * RPT
        pltpu.sync_copy(table_hbm.at[pl.ds(base, RPT)], acc.at[pl.ds(base, RPT)])
        plsc.subcore_barrier()

        sems = (sem0, sem1)

        def issue(j, b):
            pltpu.async_copy(table_hbm.at[src_v.at[j]], rows_v.at[b], sems[b])

        def wait(j, b):
            pltpu.make_async_copy(table_hbm.at[src_v.at[j]], rows_v.at[b],
                                  sems[b]).wait()

        def step(j, b):
            wait(j, b)

            def sbody(g, carry):
                wv = ew_v[j, pl.ds(g * 16, 16)]
                for k in range(16):
                    w = wv[k]
                    i = g * 16 + k
                    for f in range(F // 16):
                        sl = pl.ds(f * 16, 16)
                        rows_v[b, i, sl] = rows_v[b, i, sl] * w
                return carry

            lax.fori_loop(0, KB // 16, sbody, 0)
            pltpu.sync_copy(rows_v.at[b], acc.at[dst_v.at[j]], add=True)

            @pl.when(j + 2 < NCHUNK)
            def _():
                issue(j + 2, b)

        issue(0, 0)
        issue(1, 1)

        def body2(t, carry):
            step(2 * t, 0)
            step(2 * t + 1, 1)
            return carry

        lax.fori_loop(0, NCHUNK // 2, body2, 0)
        plsc.subcore_barrier()
        pltpu.sync_copy(acc.at[pl.ds(base, RPT)],
                        out_hbm.at[c, pl.ds(base, RPT)])

    return functools.partial(
        pl.kernel,
        out_type=jax.ShapeDtypeStruct((2, NACC, F), jnp.float32),
        mesh=_sc_mesh(),
        scratch_types=[
            pltpu.VMEM((NCHUNK, KB), jnp.int32),
            pltpu.VMEM((NCHUNK, KB), jnp.int32),
            pltpu.VMEM((NCHUNK, KB), jnp.float32),
            pltpu.VMEM((2, KB, F), jnp.float32),
            pltpu.VMEM_SHARED((NACC, F), jnp.float32),
            pltpu.SemaphoreType.DMA,
            pltpu.SemaphoreType.DMA,
        ],
        compiler_params=pltpu.CompilerParams(use_tc_tiling_on_sc=False),
    )(body)


_agg16 = _make_agg(H)
_agg64 = _make_agg(C)


def _mm1_body(x_ref, w_ref, d0_ref, d1_ref, xs_ref, dinv_ref):
    dinv = lax.rsqrt(1.0 + d0_ref[...] + d1_ref[...])
    xw = jnp.dot(x_ref[...], w_ref[...], preferred_element_type=jnp.float32)
    xs_ref[...] = xw * dinv
    dinv_ref[...] = dinv


def _tc_stage1(xpad, W1, d0, d1):
    grid = (NACC // BLK,)
    return pl.pallas_call(
        _mm1_body,
        grid=grid,
        in_specs=[
            pl.BlockSpec((BLK, D), lambda i: (i, 0)),
            pl.BlockSpec((D, H), lambda i: (0, 0)),
            pl.BlockSpec((BLK, 1), lambda i: (i, 0)),
            pl.BlockSpec((BLK, 1), lambda i: (i, 0)),
        ],
        out_specs=[
            pl.BlockSpec((BLK, H), lambda i: (i, 0)),
            pl.BlockSpec((BLK, 1), lambda i: (i, 0)),
        ],
        out_shape=[
            jax.ShapeDtypeStruct((NACC, H), jnp.float32),
            jax.ShapeDtypeStruct((NACC, 1), jnp.float32),
        ],
    )(xpad, W1, d0, d1)


def _mm2_body(p0_ref, p1_ref, xs1_ref, dinv_ref, b1_ref, w2_ref, xs2_ref):
    dinv = dinv_ref[...]
    h = (p0_ref[...] + p1_ref[...] - xs1_ref[...]) * dinv + b1_ref[...]
    h = jnp.maximum(h, 0.0)
    xw2 = jnp.dot(h, w2_ref[...], preferred_element_type=jnp.float32)
    xs2_ref[...] = xw2 * dinv


def _tc_stage2(p0, p1, xs1, dinv, b1, W2):
    grid = (NACC // BLK,)
    return pl.pallas_call(
        _mm2_body,
        grid=grid,
        in_specs=[
            pl.BlockSpec((BLK, H), lambda i: (i, 0)),
            pl.BlockSpec((BLK, H), lambda i: (i, 0)),
            pl.BlockSpec((BLK, H), lambda i: (i, 0)),
            pl.BlockSpec((BLK, 1), lambda i: (i, 0)),
            pl.BlockSpec((1, H), lambda i: (0, 0)),
            pl.BlockSpec((H, C), lambda i: (0, 0)),
        ],
        out_specs=pl.BlockSpec((BLK, C), lambda i: (i, 0)),
        out_shape=jax.ShapeDtypeStruct((NACC, C), jnp.float32),
    )(p0, p1, xs1, dinv, b1, W2)


def _final_body(q0_ref, q1_ref, xs2_ref, dinv_ref, b2_ref, o_ref):
    o = (q0_ref[...] + q1_ref[...] - xs2_ref[...]) * dinv_ref[...] + b2_ref[...]
    m = jnp.max(o, axis=1, keepdims=True)
    ex = jnp.exp(o - m)
    sden = jnp.sum(ex, axis=1, keepdims=True)
    o_ref[...] = o - m - jnp.log(sden)


def _tc_final(q0, q1, xs2, dinv, b2):
    grid = (NACC // BLK,)
    return pl.pallas_call(
        _final_body,
        grid=grid,
        in_specs=[
            pl.BlockSpec((BLK, C), lambda i: (i, 0)),
            pl.BlockSpec((BLK, C), lambda i: (i, 0)),
            pl.BlockSpec((BLK, C), lambda i: (i, 0)),
            pl.BlockSpec((BLK, 1), lambda i: (i, 0)),
            pl.BlockSpec((1, C), lambda i: (0, 0)),
        ],
        out_specs=pl.BlockSpec((BLK, C), lambda i: (i, 0)),
        out_shape=jax.ShapeDtypeStruct((NACC, C), jnp.float32),
    )(q0, q1, xs2, dinv, b2)


def kernel(x, edge_index, edge_weight, W1, b1, W2, b2):
    src = edge_index[0]
    dst = edge_index[1]
    ew = edge_weight

    pad = EPAD - E
    srcb = jnp.concatenate([src, jnp.full((pad,), N, jnp.int32)]).reshape(NT, NCHUNK, KB)
    dstb = jnp.concatenate([dst, jnp.full((pad,), N, jnp.int32)]).reshape(NT, NCHUNK, KB)
    ewb = jnp.concatenate([ew, jnp.zeros((pad,), jnp.float32)]).reshape(NT, NCHUNK, KB)

    zeros_n = jnp.zeros((NACC,), jnp.float32)
    degp = _deg_kernel(ewb, dstb, zeros_n)

    xpad = jnp.zeros((NACC, D), x.dtype).at[:N].set(x)
    d0 = degp[0].reshape(NACC, 1)
    d1 = degp[1].reshape(NACC, 1)
    xs1, dinv = _tc_stage1(xpad, W1, d0, d1)

    p = _agg16(xs1, srcb, dstb, ewb)
    xs2 = _tc_stage2(p[0], p[1], xs1, dinv, b1.reshape(1, H), W2)

    q = _agg64(xs2, srcb, dstb, ewb)
    out = _tc_final(q[0], q[1], xs2, dinv, b2.reshape(1, C))
    return out[:N]
